# Initial kernel scaffold; baseline (speedup 1.0000x reference)
import jax
import jax.numpy as jnp
from jax.experimental import pallas as pl


def kernel(x, edge_index, edge_weights, gcn0_Wrel, gcn0_Wroot, gcn0_b, gcn0_gamma, gcn0_beta, gcn1_Wrel, gcn1_Wroot, gcn1_b, gcn1_gamma, gcn1_beta, mlp0_W, mlp0_b, mlp0_gamma, mlp0_beta, out_W, out_b):
    raise NotImplementedError("write your pallas kernel here")


# speedup vs baseline: 4.7637x; 4.7637x over previous
import functools

import jax
import jax.numpy as jnp
from jax import lax
from jax.experimental import pallas as pl
from jax.experimental.pallas import tpu as pltpu
from jax.experimental.pallas import tpu_sc as plsc

N = 10000
D = 128
E = 320000
LANES = 16
NCORES = 2
NSUB = 16
NWORK = NCORES * NSUB
CHUNK = 128
NCHUNKS = E // CHUNK
ZROWS = 200
NZCH = N // ZROWS


def _sc_conv_body(x_hbm, src_hbm, dst_hbm, w_hbm, out_hbm,
                  src_v, dst_v, w_v, rows_v, zero_v, agg_sh, sem):
    c = lax.axis_index("c")
    s = lax.axis_index("s")
    wid = c * NSUB + s

    def zrow(r, _):
        for j in range(D // LANES):
            zero_v[r, pl.ds(j * LANES, LANES)] = jnp.zeros((LANES,), jnp.float32)
        return _
    lax.fori_loop(0, ZROWS, zrow, 0, unroll=False)
    for t in range((NZCH + NSUB - 1) // NSUB):
        k = s + t * NSUB

        @pl.when(k < NZCH)
        def _():
            pltpu.sync_copy(zero_v, agg_sh.at[pl.ds(k * ZROWS, ZROWS)])
    plsc.subcore_barrier()

    lo = (wid * NCHUNKS) // NWORK
    hi = ((wid + 1) * NCHUNKS) // NWORK

    def chunk_body(i, _):
        base = i * CHUNK
        pltpu.sync_copy(src_hbm.at[pl.ds(base, CHUNK)], src_v)
        pltpu.sync_copy(dst_hbm.at[pl.ds(base, CHUNK)], dst_v)
        pltpu.sync_copy(w_hbm.at[pl.ds(base, CHUNK)], w_v)
        pltpu.async_copy(x_hbm.at[src_v], rows_v, sem).wait()

        def scale(g, __):
            wblk = w_v[pl.ds(g * LANES, LANES)]
            for l in range(LANES):
                lane = jnp.full((LANES,), l, jnp.int32)
                wv = wblk.at[lane].get(mode="promise_in_bounds")
                e = g * LANES + l
                for j in range(D // LANES):
                    sl = pl.ds(j * LANES, LANES)
                    rows_v[e, sl] = rows_v[e, sl] * wv
            return __
        lax.fori_loop(0, CHUNK // LANES, scale, 0, unroll=False)

        pltpu.sync_copy(rows_v, agg_sh.at[dst_v], add=True)
        return _
    lax.fori_loop(lo, hi, chunk_body, 0, unroll=False)

    plsc.subcore_barrier()

    for t in range((NZCH + NSUB - 1) // NSUB):
        k = s + t * NSUB

        @pl.when(k < NZCH)
        def _():
            pltpu.sync_copy(agg_sh.at[pl.ds(k * ZROWS, ZROWS)],
                            out_hbm.at[c, pl.ds(k * ZROWS, ZROWS)])


@functools.cache
def _make_sc_conv():
    return pl.kernel(
        _sc_conv_body,
        out_type=jax.ShapeDtypeStruct((NCORES, N, D), jnp.float32),
        mesh=plsc.VectorSubcoreMesh(core_axis_name="c", subcore_axis_name="s"),
        scratch_types=[
            pltpu.VMEM((CHUNK,), jnp.int32),
            pltpu.VMEM((CHUNK,), jnp.int32),
            pltpu.VMEM((CHUNK,), jnp.float32),
            pltpu.VMEM((CHUNK, D), jnp.float32),
            pltpu.VMEM((ZROWS, D), jnp.float32),
            pltpu.VMEM_SHARED((N, D), jnp.float32),
            pltpu.SemaphoreType.DMA,
        ],
    )


def _sc_conv(x, src, dst, w):
    return _make_sc_conv()(x, src, dst, w)


def _dotT(a, b):
    return lax.dot_general(a, b, (((1,), (1,)), ((), ())),
                           preferred_element_type=jnp.float32)


def _bn_relu(z, gamma, beta):
    mu = jnp.mean(z, axis=0, keepdims=True)
    var = jnp.mean((z - mu) ** 2, axis=0, keepdims=True)
    h = (z - mu) / jnp.sqrt(var + 1e-5) * gamma + beta
    return jnp.maximum(h, 0.0)


def _tc0_body(agg2_ref, x_ref, wrel_ref, wroot_ref, b_ref, g_ref, be_ref, out_ref):
    agg = agg2_ref[0] + agg2_ref[1]
    z = _dotT(agg, wrel_ref[...]) + _dotT(x_ref[...], wroot_ref[...]) + b_ref[...]
    out_ref[...] = _bn_relu(z, g_ref[...], be_ref[...])


def _tc1_body(agg2_ref, h0_ref, wrel_ref, wroot_ref, b_ref, g_ref, be_ref,
              mlpw_ref, mlpb_ref, g2_ref, be2_ref, outw_ref, outb_ref, out_ref):
    agg = agg2_ref[0] + agg2_ref[1]
    z = _dotT(agg, wrel_ref[...]) + _dotT(h0_ref[...], wroot_ref[...]) + b_ref[...]
    h1 = _bn_relu(z, g_ref[...], be_ref[...])
    z2 = _dotT(h1, mlpw_ref[...]) + mlpb_ref[...]
    h2 = _bn_relu(z2, g2_ref[...], be2_ref[...])
    o = _dotT(h2, outw_ref[...]) + outb_ref[...]
    out_ref[...] = jax.nn.sigmoid(o)


def _tc0(agg2, x, wrel, wroot, b, g, be):
    return pl.pallas_call(
        _tc0_body,
        out_shape=jax.ShapeDtypeStruct((N, D), jnp.float32),
    )(agg2, x, wrel, wroot, b.reshape(1, D), g.reshape(1, D), be.reshape(1, D))


def _tc1(agg2, h0, wrel, wroot, b, g, be, mlpw, mlpb, g2, be2, outw, outb):
    outw_wide = jnp.broadcast_to(outw, (D, D))
    outb_wide = jnp.broadcast_to(outb.reshape(1, 1), (1, D))
    return pl.pallas_call(
        _tc1_body,
        out_shape=jax.ShapeDtypeStruct((N, D), jnp.float32),
    )(agg2, h0, wrel, wroot, b.reshape(1, D), g.reshape(1, D), be.reshape(1, D),
      mlpw, mlpb.reshape(1, D), g2.reshape(1, D), be2.reshape(1, D),
      outw_wide, outb_wide)


def kernel(x, edge_index, edge_weights,
           gcn0_Wrel, gcn0_Wroot, gcn0_b, gcn0_gamma, gcn0_beta,
           gcn1_Wrel, gcn1_Wroot, gcn1_b, gcn1_gamma, gcn1_beta,
           mlp0_W, mlp0_b, mlp0_gamma, mlp0_beta,
           out_W, out_b):
    src = edge_index[0]
    dst = edge_index[1]
    agg0 = _sc_conv(x, src, dst, edge_weights)
    h0 = _tc0(agg0, x, gcn0_Wrel, gcn0_Wroot, gcn0_b, gcn0_gamma, gcn0_beta)
    agg1 = _sc_conv(h0, src, dst, edge_weights)
    wide = _tc1(agg1, h0, gcn1_Wrel, gcn1_Wroot, gcn1_b, gcn1_gamma, gcn1_beta,
                mlp0_W, mlp0_b, mlp0_gamma, mlp0_beta, out_W, out_b)
    return wide[:, :1]


# speedup vs baseline: 8.8841x; 1.8649x over previous
import functools

import jax
import jax.numpy as jnp
from jax import lax
from jax.experimental import pallas as pl
from jax.experimental.pallas import tpu as pltpu
from jax.experimental.pallas import tpu_sc as plsc

N = 10000
D = 128
E = 320000
LANES = 16
NCORES = 2
NSUB = 16
NWORK = NCORES * NSUB
EPW = E // NWORK
CH = 80
NCH = EPW // CH
ZROWS = 40
NZCH = N // ZROWS


def _sc_conv_body(x_hbm, src_hbm, dst_hbm, w_hbm, out_hbm,
                  src_all, w_all, dst0, dst1, rows0, rows1, zero_v, agg_sh,
                  g0, g1, s0, s1, d0, d1):
    c = lax.axis_index("c")
    s = lax.axis_index("s")
    wid = c * NSUB + s
    ebase = wid * EPW

    def zrow(r, _):
        for j in range(D // LANES):
            zero_v[r, pl.ds(j * LANES, LANES)] = jnp.zeros((LANES,), jnp.float32)
        return _
    lax.fori_loop(0, ZROWS, zrow, 0, unroll=False)

    pltpu.sync_copy(src_hbm.at[pl.ds(ebase, EPW)], src_all)
    pltpu.sync_copy(w_hbm.at[pl.ds(ebase, EPW)], w_all)

    for t in range((NZCH + NSUB - 1) // NSUB):
        k = s + t * NSUB

        @pl.when(k < NZCH)
        def _():
            pltpu.sync_copy(zero_v, agg_sh.at[pl.ds(k * ZROWS, ZROWS)])
    plsc.subcore_barrier()

    def scale_rows(rows, i):
        def scale(g, __):
            wblk = w_all[pl.ds(i * CH + g * LANES, LANES)]
            for l in range(LANES):
                lane = jnp.full((LANES,), l, jnp.int32)
                wv = wblk.at[lane].get(mode="promise_in_bounds")
                e = g * LANES + l
                for j in range(D // LANES):
                    sl = pl.ds(j * LANES, LANES)
                    rows[e, sl] = rows[e, sl] * wv
            return __
        lax.fori_loop(0, CH // LANES, scale, 0, unroll=False)

    def step(i, mine, g_mine, s_mine, dst_mine, d_mine,
             other, g_other, s_other, dst_other, d_other):
        pltpu.make_async_copy(x_hbm.at[src_all.at[pl.ds(i * CH, CH)]],
                              mine, g_mine).wait()
        pltpu.make_async_copy(dst_hbm.at[pl.ds(ebase + i * CH, CH)],
                              dst_mine, d_mine).wait()

        @pl.when(i >= 1)
        def _():
            pltpu.make_async_copy(other, agg_sh.at[dst_other], s_other).wait()

        @pl.when(i + 1 < NCH)
        def _():
            pltpu.async_copy(x_hbm.at[src_all.at[pl.ds((i + 1) * CH, CH)]],
                             other, g_other)
            pltpu.async_copy(dst_hbm.at[pl.ds(ebase + (i + 1) * CH, CH)],
                             dst_other, d_other)

        scale_rows(mine, i)
        pltpu.async_copy(mine, agg_sh.at[dst_mine], s_mine, add=True)

    pltpu.async_copy(x_hbm.at[src_all.at[pl.ds(0, CH)]], rows0, g0)
    pltpu.async_copy(dst_hbm.at[pl.ds(ebase, CH)], dst0, d0)

    def chunk_body(i, carry):
        @pl.when(i % 2 == 0)
        def _():
            step(i, rows0, g0, s0, dst0, d0, rows1, g1, s1, dst1, d1)

        @pl.when(i % 2 == 1)
        def _():
            step(i, rows1, g1, s1, dst1, d1, rows0, g0, s0, dst0, d0)
        return carry
    lax.fori_loop(0, NCH, chunk_body, 0, unroll=False)

    pltpu.make_async_copy(rows0, agg_sh.at[dst0], s0).wait()

    plsc.subcore_barrier()

    for t in range((NZCH + NSUB - 1) // NSUB):
        k = s + t * NSUB

        @pl.when(k < NZCH)
        def _():
            pltpu.sync_copy(agg_sh.at[pl.ds(k * ZROWS, ZROWS)],
                            out_hbm.at[c, pl.ds(k * ZROWS, ZROWS)])


@functools.cache
def _make_sc_conv():
    return pl.kernel(
        _sc_conv_body,
        out_type=jax.ShapeDtypeStruct((NCORES, N, D), jnp.float32),
        mesh=plsc.VectorSubcoreMesh(core_axis_name="c", subcore_axis_name="s"),
        scratch_types=[
            pltpu.VMEM((EPW,), jnp.int32),
            pltpu.VMEM((EPW,), jnp.float32),
            pltpu.VMEM((CH,), jnp.int32),
            pltpu.VMEM((CH,), jnp.int32),
            pltpu.VMEM((CH, D), jnp.float32),
            pltpu.VMEM((CH, D), jnp.float32),
            pltpu.VMEM((ZROWS, D), jnp.float32),
            pltpu.VMEM_SHARED((N, D), jnp.float32),
            pltpu.SemaphoreType.DMA,
            pltpu.SemaphoreType.DMA,
            pltpu.SemaphoreType.DMA,
            pltpu.SemaphoreType.DMA,
            pltpu.SemaphoreType.DMA,
            pltpu.SemaphoreType.DMA,
        ],
    )


def _sc_conv(x, src, dst, w):
    return _make_sc_conv()(x, src, dst, w)


def _dotT(a, b):
    return lax.dot_general(a, b, (((1,), (1,)), ((), ())),
                           preferred_element_type=jnp.float32)


def _bn_relu(z, gamma, beta):
    mu = jnp.mean(z, axis=0, keepdims=True)
    var = jnp.mean((z - mu) ** 2, axis=0, keepdims=True)
    h = (z - mu) / jnp.sqrt(var + 1e-5) * gamma + beta
    return jnp.maximum(h, 0.0)


def _tc0_body(agg2_ref, x_ref, wrel_ref, wroot_ref, b_ref, g_ref, be_ref, out_ref):
    agg = agg2_ref[0] + agg2_ref[1]
    z = _dotT(agg, wrel_ref[...]) + _dotT(x_ref[...], wroot_ref[...]) + b_ref[...]
    out_ref[...] = _bn_relu(z, g_ref[...], be_ref[...])


def _tc1_body(agg2_ref, h0_ref, wrel_ref, wroot_ref, b_ref, g_ref, be_ref,
              mlpw_ref, mlpb_ref, g2_ref, be2_ref, outw_ref, outb_ref, out_ref):
    agg = agg2_ref[0] + agg2_ref[1]
    z = _dotT(agg, wrel_ref[...]) + _dotT(h0_ref[...], wroot_ref[...]) + b_ref[...]
    h1 = _bn_relu(z, g_ref[...], be_ref[...])
    z2 = _dotT(h1, mlpw_ref[...]) + mlpb_ref[...]
    h2 = _bn_relu(z2, g2_ref[...], be2_ref[...])
    o = _dotT(h2, outw_ref[...]) + outb_ref[...]
    out_ref[...] = jax.nn.sigmoid(o)


def _tc0(agg2, x, wrel, wroot, b, g, be):
    return pl.pallas_call(
        _tc0_body,
        out_shape=jax.ShapeDtypeStruct((N, D), jnp.float32),
    )(agg2, x, wrel, wroot, b.reshape(1, D), g.reshape(1, D), be.reshape(1, D))


def _tc1(agg2, h0, wrel, wroot, b, g, be, mlpw, mlpb, g2, be2, outw, outb):
    outw_wide = jnp.broadcast_to(outw, (D, D))
    outb_wide = jnp.broadcast_to(outb.reshape(1, 1), (1, D))
    return pl.pallas_call(
        _tc1_body,
        out_shape=jax.ShapeDtypeStruct((N, D), jnp.float32),
    )(agg2, h0, wrel, wroot, b.reshape(1, D), g.reshape(1, D), be.reshape(1, D),
      mlpw, mlpb.reshape(1, D), g2.reshape(1, D), be2.reshape(1, D),
      outw_wide, outb_wide)


def kernel(x, edge_index, edge_weights,
           gcn0_Wrel, gcn0_Wroot, gcn0_b, gcn0_gamma, gcn0_beta,
           gcn1_Wrel, gcn1_Wroot, gcn1_b, gcn1_gamma, gcn1_beta,
           mlp0_W, mlp0_b, mlp0_gamma, mlp0_beta,
           out_W, out_b):
    src = edge_index[0]
    dst = edge_index[1]
    agg0 = _sc_conv(x, src, dst, edge_weights)
    h0 = _tc0(agg0, x, gcn0_Wrel, gcn0_Wroot, gcn0_b, gcn0_gamma, gcn0_beta)
    agg1 = _sc_conv(h0, src, dst, edge_weights)
    wide = _tc1(agg1, h0, gcn1_Wrel, gcn1_Wroot, gcn1_b, gcn1_gamma, gcn1_beta,
                mlp0_W, mlp0_b, mlp0_gamma, mlp0_beta, out_W, out_b)
    return wide[:, :1]
